# baseline (speedup 1.0000x reference)
import functools

import jax
import jax.numpy as jnp
from jax.experimental import pallas as pl
from jax.experimental.pallas import tpu as pltpu

_H = 16
_GS = 128
_NLAB = 16
_RB, _RMIN, _RMAX = 0.5, 0.25, 0.75
_LP, _LV = 0.25, 0.25


def _qkv_kernel(x_ref, w_ref, o_ref):
    o_ref[0] = jnp.dot(x_ref[0], w_ref[...], preferred_element_type=jnp.float32)


def _attn_kernel(q_ref, klo_ref, khi_ref, kg_ref, vlo_ref, vhi_ref, vg_ref,
                 lablo_ref, labhi_ref, labq_ref, qs_ref, wpt_ref,
                 out_ref, asp_ref, *, hhalf, d, gs, nlab):
    hh = pl.program_id(1)
    k2 = 2 * gs
    scale = d ** (-0.5)

    ql_col = labq_ref[0]
    klab = jnp.concatenate([lablo_ref[0], labhi_ref[0]], axis=-1)
    mask = ql_col == klab

    ql_row = lablo_ref[0]
    lab_iota = jax.lax.broadcasted_iota(jnp.int32, (nlab, gs), 0)
    counts = jnp.sum((lab_iota == ql_row).astype(jnp.float32), axis=1)
    purity = jnp.max(counts) * (1.0 / gs)
    qs = qs_ref[0]
    mu = jnp.mean(qs)
    svar = jnp.mean((qs - mu) * (qs - mu))
    focus = jnp.clip(_RB + _LP * purity - _LV * svar, _RMIN, _RMAX)
    keep = jnp.clip(jnp.ceil(focus * k2).astype(jnp.int32), 1, k2)

    rowcnt = jnp.sum(mask.astype(jnp.int32), axis=-1, keepdims=True)
    need_search = jnp.any(rowcnt > keep)
    bias = jnp.where(mask, 0.0, -10000.0)
    lane = jax.lax.broadcasted_iota(jnp.int32, (1, k2), 1)

    def _slow(attn):
        bits = jax.lax.bitcast_convert_type(attn, jnp.int32)
        lo0 = jnp.zeros((gs, 1), jnp.int32)
        hi0 = jnp.full((gs, 1), 0x3F800001, jnp.int32)

        def vbody(_, carry):
            lo, hi = carry
            mid = (lo + hi) // 2
            cnt = jnp.sum((bits >= mid).astype(jnp.int32), axis=-1,
                          keepdims=True)
            pred = cnt >= keep
            return jnp.where(pred, mid, lo), jnp.where(pred, hi, mid)

        vlo, _ = jax.lax.fori_loop(0, 31, vbody, (lo0, hi0))

        gt = bits > vlo
        c_gt = jnp.sum(gt.astype(jnp.int32), axis=-1, keepdims=True)
        budget = keep - c_gt
        eq = bits == vlo
        eqi = eq.astype(jnp.int32)

        tlo0 = jnp.zeros((gs, 1), jnp.int32)
        thi0 = jnp.full((gs, 1), k2 + 1, jnp.int32)

        def tbody(_, carry):
            tlo, thi = carry
            mid = (tlo + thi) // 2
            cnt = jnp.sum(jnp.where(lane < mid, eqi, 0), axis=-1,
                          keepdims=True)
            pred = cnt <= budget
            return jnp.where(pred, mid, tlo), jnp.where(pred, thi, mid)

        tlo, _ = jax.lax.fori_loop(0, 10, tbody, (tlo0, thi0))

        smask = gt | (eq & (lane < tlo))
        asp = jnp.where(smask, attn, 0.0)
        return asp / (jnp.sum(asp, axis=-1, keepdims=True) + 1e-9)

    dimnums = (((1,), (1,)), ((), ()))
    outs = []
    for hl in range(hhalf):
        c = hl * d
        qh = q_ref[0, :, c:c + d]
        lg_lo = jax.lax.dot_general(qh, klo_ref[0, :, c:c + d], dimnums,
                                    preferred_element_type=jnp.float32)
        lg_hi = jax.lax.dot_general(qh, khi_ref[0, :, c:c + d], dimnums,
                                    preferred_element_type=jnp.float32)
        ml = jnp.concatenate([lg_lo, lg_hi], axis=1) * scale + bias
        mx = jnp.max(ml, axis=-1, keepdims=True)
        e = jnp.exp(ml - mx)
        attn = e * (1.0 / jnp.sum(e, axis=-1, keepdims=True))
        a_h = jax.lax.cond(need_search, _slow, lambda a: a, attn)
        asp_ref[0, hl] = a_h
        o_h = jnp.dot(a_h[:, :gs], vlo_ref[0, :, c:c + d],
                      preferred_element_type=jnp.float32)
        o_h = o_h + jnp.dot(a_h[:, gs:], vhi_ref[0, :, c:c + d],
                            preferred_element_type=jnp.float32)
        glog = jax.lax.dot_general(qh, kg_ref[0, :, c:c + d], dimnums,
                                   preferred_element_type=jnp.float32) * scale
        gmx = jnp.max(glog, axis=-1, keepdims=True)
        ge = jnp.exp(glog - gmx)
        gsm = ge * (1.0 / jnp.sum(ge, axis=-1, keepdims=True))
        o_h = o_h + jnp.dot(gsm, vg_ref[0, :, c:c + d],
                            preferred_element_type=jnp.float32)
        outs.append(o_h)
    out_half = jnp.concatenate(outs, axis=1)
    partial = jnp.dot(out_half, wpt_ref[...],
                      preferred_element_type=jnp.float32)

    @pl.when(hh == 0)
    def _():
        out_ref[0] = partial

    @pl.when(hh != 0)
    def _():
        out_ref[0] = out_ref[0] + partial


@jax.jit
def kernel(x, labels, scores, prototypes, Wq, Wk, Wv, Wproj):
    B, N, C = x.shape
    QK = Wq.shape[0]
    M = prototypes.shape[1]
    h, gs, d = _H, _GS, QK // _H
    dv = C // h
    ng = N // gs
    k2 = 2 * gs
    hhalf = h // 2

    tail = jnp.flip(x[:, -gs:], axis=1)
    stream = jnp.concatenate([x, tail, prototypes], axis=1)
    NP = ((N + gs + M + 255) // 256) * 256
    stream = jnp.pad(stream, ((0, 0), (0, NP - (N + gs + M)), (0, 0)))
    w_all = jnp.concatenate([Wq, Wk, Wv], axis=0).T
    W3 = w_all.shape[1]

    qkv = pl.pallas_call(
        _qkv_kernel,
        grid=(B, NP // 256),
        in_specs=[
            pl.BlockSpec((1, 256, C), lambda b, i: (b, i, 0)),
            pl.BlockSpec((C, W3), lambda b, i: (0, 0)),
        ],
        out_specs=pl.BlockSpec((1, 256, W3), lambda b, i: (b, i, 0)),
        out_shape=jax.ShapeDtypeStruct((B, NP, W3), jnp.float32),
        compiler_params=pltpu.CompilerParams(
            dimension_semantics=("parallel", "parallel")),
    )(stream, w_all)

    labels = labels.astype(jnp.int32)
    lab_tail = jnp.flip(labels[:, -gs:], axis=1)
    labp = jnp.concatenate([labels, lab_tail], axis=1).reshape(B * (ng + 1), 1, gs)
    labq = labels.reshape(B * ng, gs, 1)
    scr = scores.reshape(B * ng, 1, gs).astype(jnp.float32)
    wpt = Wproj.T

    pg = (N + gs) // M
    hd = hhalf * d
    cq, ck, cv = 0, QK // hd, (QK + QK) // hd
    in_specs = [
        pl.BlockSpec((1, gs, hd), lambda i, hh: (i // ng, i % ng, cq + hh)),
        pl.BlockSpec((1, gs, hd), lambda i, hh: (i // ng, i % ng, ck + hh)),
        pl.BlockSpec((1, gs, hd), lambda i, hh: (i // ng, (i % ng) + 1, ck + hh)),
        pl.BlockSpec((1, M, hd), lambda i, hh: (i // ng, pg, ck + hh)),
        pl.BlockSpec((1, gs, hd), lambda i, hh: (i // ng, i % ng, cv + hh)),
        pl.BlockSpec((1, gs, hd), lambda i, hh: (i // ng, (i % ng) + 1, cv + hh)),
        pl.BlockSpec((1, M, hd), lambda i, hh: (i // ng, pg, cv + hh)),
        pl.BlockSpec((1, 1, gs), lambda i, hh: ((i // ng) * (ng + 1) + i % ng, 0, 0)),
        pl.BlockSpec((1, 1, gs), lambda i, hh: ((i // ng) * (ng + 1) + i % ng + 1, 0, 0)),
        pl.BlockSpec((1, gs, 1), lambda i, hh: (i, 0, 0)),
        pl.BlockSpec((1, 1, gs), lambda i, hh: (i, 0, 0)),
        pl.BlockSpec((hd, C), lambda i, hh: (hh, 0)),
    ]
    out_specs = [
        pl.BlockSpec((1, gs, C), lambda i, hh: (i // ng, i % ng, 0)),
        pl.BlockSpec((1, hhalf, gs, k2), lambda i, hh: (i, hh, 0, 0)),
    ]
    out_shape = [
        jax.ShapeDtypeStruct((B, N, C), jnp.float32),
        jax.ShapeDtypeStruct((B * ng, h, gs, k2), jnp.float32),
    ]

    body = functools.partial(_attn_kernel, hhalf=hhalf, d=d, gs=gs, nlab=_NLAB)
    out, asp = pl.pallas_call(
        body,
        grid=(B * ng, 2),
        in_specs=in_specs,
        out_specs=out_specs,
        out_shape=out_shape,
        compiler_params=pltpu.CompilerParams(
            dimension_semantics=("parallel", "arbitrary")),
    )(qkv, qkv, qkv, qkv, qkv, qkv, qkv, labp, labp, labq, scr, wpt)

    return out, asp.reshape(B, ng, h, gs, k2)


# speedup vs baseline: 1.4309x; 1.4309x over previous
import functools

import jax
import jax.numpy as jnp
from jax.experimental import pallas as pl
from jax.experimental.pallas import tpu as pltpu

_H = 16
_GS = 128
_NLAB = 16
_RB, _RMIN, _RMAX = 0.5, 0.25, 0.75
_LP, _LV = 0.25, 0.25


def _qkv_kernel(x_ref, w_ref, o_ref):
    o_ref[0] = jnp.dot(x_ref[0], w_ref[...], preferred_element_type=jnp.float32)


def _attn_kernel(q_ref, klo_ref, khi_ref, kg_ref, vlo_ref, vhi_ref, vg_ref,
                 lablo_ref, labhi_ref, labq_ref, qs_ref, wpt_ref,
                 out_ref, asp_ref, *, h, d, gs, nlab):
    k2 = 2 * gs
    scale = d ** (-0.5)

    ql_col = labq_ref[0]
    klab = jnp.concatenate([lablo_ref[0], labhi_ref[0]], axis=-1)
    mask = ql_col == klab

    ql_row = lablo_ref[0]
    lab_iota = jax.lax.broadcasted_iota(jnp.int32, (nlab, gs), 0)
    counts = jnp.sum((lab_iota == ql_row).astype(jnp.float32), axis=1)
    purity = jnp.max(counts) * (1.0 / gs)
    qs = qs_ref[0]
    mu = jnp.mean(qs)
    svar = jnp.mean((qs - mu) * (qs - mu))
    focus = jnp.clip(_RB + _LP * purity - _LV * svar, _RMIN, _RMAX)
    keep = jnp.clip(jnp.ceil(focus * k2).astype(jnp.int32), 1, k2)

    rowcnt = jnp.sum(mask.astype(jnp.float32), axis=-1, keepdims=True)
    need_search = jnp.any(rowcnt > keep.astype(jnp.float32))
    bias = jnp.where(mask, 0.0, -10000.0)

    dimnums = (((1,), (1,)), ((), ()))
    for hl in range(h):
        c = hl * d
        qh = q_ref[0, :, c:c + d]
        lg_lo = jax.lax.dot_general(qh, klo_ref[0, :, c:c + d], dimnums,
                                    preferred_element_type=jnp.float32)
        lg_hi = jax.lax.dot_general(qh, khi_ref[0, :, c:c + d], dimnums,
                                    preferred_element_type=jnp.float32)
        ml = jnp.concatenate([lg_lo, lg_hi], axis=1) * scale + bias
        mx = jnp.max(ml, axis=-1, keepdims=True)
        e = jnp.exp(ml - mx)
        asp_ref[0, hl] = e * (1.0 / jnp.sum(e, axis=-1, keepdims=True))

    @pl.when(need_search)
    def _fixup():
        attn = jnp.reshape(asp_ref[0], (h * gs, k2))
        bits = jax.lax.bitcast_convert_type(attn, jnp.int32)
        lo0 = jnp.zeros((h * gs, 1), jnp.int32)
        hi0 = jnp.full((h * gs, 1), 0x3F800001, jnp.int32)

        def vbody(_, carry):
            lo, hi = carry
            mid = (lo + hi) // 2
            cnt = jnp.sum((bits >= mid).astype(jnp.int32), axis=-1,
                          keepdims=True)
            pred = cnt >= keep
            return jnp.where(pred, mid, lo), jnp.where(pred, hi, mid)

        vlo, _ = jax.lax.fori_loop(0, 31, vbody, (lo0, hi0))

        gt = bits > vlo
        c_gt = jnp.sum(gt.astype(jnp.int32), axis=-1, keepdims=True)
        budget = keep - c_gt
        eq = bits == vlo
        eqi = eq.astype(jnp.int32)
        lane = jax.lax.broadcasted_iota(jnp.int32, (1, k2), 1)

        tlo0 = jnp.zeros((h * gs, 1), jnp.int32)
        thi0 = jnp.full((h * gs, 1), k2 + 1, jnp.int32)

        def tbody(_, carry):
            tlo, thi = carry
            mid = (tlo + thi) // 2
            cnt = jnp.sum(jnp.where(lane < mid, eqi, 0), axis=-1,
                          keepdims=True)
            pred = cnt <= budget
            return jnp.where(pred, mid, tlo), jnp.where(pred, thi, mid)

        tlo, _ = jax.lax.fori_loop(0, 10, tbody, (tlo0, thi0))

        smask = gt | (eq & (lane < tlo))
        asp = jnp.where(smask, attn, 0.0)
        asp = asp / (jnp.sum(asp, axis=-1, keepdims=True) + 1e-9)
        asp_ref[0] = jnp.reshape(asp, (h, gs, k2))

    outs = []
    for hl in range(h):
        c = hl * d
        a_h = asp_ref[0, hl]
        o_h = jnp.dot(a_h[:, :gs], vlo_ref[0, :, c:c + d],
                      preferred_element_type=jnp.float32)
        o_h = o_h + jnp.dot(a_h[:, gs:], vhi_ref[0, :, c:c + d],
                            preferred_element_type=jnp.float32)
        glog = jax.lax.dot_general(q_ref[0, :, c:c + d], kg_ref[0, :, c:c + d],
                                   dimnums,
                                   preferred_element_type=jnp.float32) * scale
        gmx = jnp.max(glog, axis=-1, keepdims=True)
        ge = jnp.exp(glog - gmx)
        gsm = ge * (1.0 / jnp.sum(ge, axis=-1, keepdims=True))
        o_h = o_h + jnp.dot(gsm, vg_ref[0, :, c:c + d],
                            preferred_element_type=jnp.float32)
        outs.append(o_h)
    out_full = jnp.concatenate(outs, axis=1)
    out_ref[0] = jnp.dot(out_full, wpt_ref[...],
                         preferred_element_type=jnp.float32)


@jax.jit
def kernel(x, labels, scores, prototypes, Wq, Wk, Wv, Wproj):
    B, N, C = x.shape
    QK = Wq.shape[0]
    M = prototypes.shape[1]
    h, gs, d = _H, _GS, QK // _H
    ng = N // gs
    k2 = 2 * gs

    tail = jnp.flip(x[:, -gs:], axis=1)
    stream = jnp.concatenate([x, tail, prototypes], axis=1)
    NP = ((N + gs + M + 255) // 256) * 256
    stream = jnp.pad(stream, ((0, 0), (0, NP - (N + gs + M)), (0, 0)))
    w_all = jnp.concatenate([Wq, Wk, Wv], axis=0).T
    W3 = w_all.shape[1]

    qkv = pl.pallas_call(
        _qkv_kernel,
        grid=(B, NP // 256),
        in_specs=[
            pl.BlockSpec((1, 256, C), lambda b, i: (b, i, 0)),
            pl.BlockSpec((C, W3), lambda b, i: (0, 0)),
        ],
        out_specs=pl.BlockSpec((1, 256, W3), lambda b, i: (b, i, 0)),
        out_shape=jax.ShapeDtypeStruct((B, NP, W3), jnp.float32),
        compiler_params=pltpu.CompilerParams(
            dimension_semantics=("parallel", "parallel")),
    )(stream, w_all)

    labels = labels.astype(jnp.int32)
    lab_tail = jnp.flip(labels[:, -gs:], axis=1)
    labp = jnp.concatenate([labels, lab_tail], axis=1).reshape(B * (ng + 1), 1, gs)
    labq = labels.reshape(B * ng, gs, 1)
    scr = scores.reshape(B * ng, 1, gs).astype(jnp.float32)
    wpt = Wproj.T

    pg = (N + gs) // M

    in_specs = [
        pl.BlockSpec((1, gs, QK), lambda i: (i // ng, i % ng, 0)),
        pl.BlockSpec((1, gs, QK), lambda i: (i // ng, i % ng, 1)),
        pl.BlockSpec((1, gs, QK), lambda i: (i // ng, (i % ng) + 1, 1)),
        pl.BlockSpec((1, M, QK), lambda i: (i // ng, pg, 1)),
        pl.BlockSpec((1, gs, C), lambda i: (i // ng, i % ng, 2)),
        pl.BlockSpec((1, gs, C), lambda i: (i // ng, (i % ng) + 1, 2)),
        pl.BlockSpec((1, M, C), lambda i: (i // ng, pg, 2)),
        pl.BlockSpec((1, 1, gs), lambda i: ((i // ng) * (ng + 1) + i % ng, 0, 0)),
        pl.BlockSpec((1, 1, gs), lambda i: ((i // ng) * (ng + 1) + i % ng + 1, 0, 0)),
        pl.BlockSpec((1, gs, 1), lambda i: (i, 0, 0)),
        pl.BlockSpec((1, 1, gs), lambda i: (i, 0, 0)),
        pl.BlockSpec((C, C), lambda i: (0, 0)),
    ]
    out_specs = [
        pl.BlockSpec((1, gs, C), lambda i: (i // ng, i % ng, 0)),
        pl.BlockSpec((1, h, gs, k2), lambda i: (i, 0, 0, 0)),
    ]
    out_shape = [
        jax.ShapeDtypeStruct((B, N, C), jnp.float32),
        jax.ShapeDtypeStruct((B * ng, h, gs, k2), jnp.float32),
    ]

    body = functools.partial(_attn_kernel, h=h, d=d, gs=gs, nlab=_NLAB)
    out, asp = pl.pallas_call(
        body,
        grid=(B * ng,),
        in_specs=in_specs,
        out_specs=out_specs,
        out_shape=out_shape,
        compiler_params=pltpu.CompilerParams(
            dimension_semantics=("parallel",)),
    )(qkv, qkv, qkv, qkv, qkv, qkv, qkv, labp, labp, labq, scr, wpt)

    return out, asp.reshape(B, ng, h, gs, k2)


# speedup vs baseline: 1.5408x; 1.0768x over previous
import functools

import jax
import jax.numpy as jnp
from jax.experimental import pallas as pl
from jax.experimental.pallas import tpu as pltpu

_H = 16
_GS = 128
_NLAB = 16
_RB, _RMIN, _RMAX = 0.5, 0.25, 0.75
_LP, _LV = 0.25, 0.25


def _qkv_kernel(x_ref, w_ref, o_ref):
    o_ref[0] = jnp.dot(x_ref[0], w_ref[...], preferred_element_type=jnp.float32)


def _attn_kernel(q_ref, klo_ref, khi_ref, kg_ref, vlo_ref, vhi_ref, vg_ref,
                 lablo_ref, labhi_ref, labq_ref, qs_ref, wpt_ref,
                 out_ref, asp_ref, oav_ref, ogl_ref, *, h, d, gs, nlab):
    k2 = 2 * gs
    scale = d ** (-0.5)

    ql_col = labq_ref[0]
    klab = jnp.concatenate([lablo_ref[0], labhi_ref[0]], axis=-1)
    mask = ql_col == klab

    ql_row = lablo_ref[0]
    lab_iota = jax.lax.broadcasted_iota(jnp.int32, (nlab, gs), 0)
    counts = jnp.sum((lab_iota == ql_row).astype(jnp.float32), axis=1)
    purity = jnp.max(counts) * (1.0 / gs)
    qs = qs_ref[0]
    mu = jnp.mean(qs)
    svar = jnp.mean((qs - mu) * (qs - mu))
    focus = jnp.clip(_RB + _LP * purity - _LV * svar, _RMIN, _RMAX)
    keep = jnp.clip(jnp.ceil(focus * k2).astype(jnp.int32), 1, k2)

    rowcnt = jnp.sum(mask.astype(jnp.float32), axis=-1, keepdims=True)
    need_search = jnp.any(rowcnt > keep.astype(jnp.float32))
    bias = jnp.where(mask, 0.0, -10000.0)

    dimnums = (((1,), (1,)), ((), ()))

    def _av(a_h, c):
        o_h = jnp.dot(a_h[:, :gs], vlo_ref[0, :, c:c + d],
                      preferred_element_type=jnp.float32)
        return o_h + jnp.dot(a_h[:, gs:], vhi_ref[0, :, c:c + d],
                             preferred_element_type=jnp.float32)

    for hl in range(h):
        c = hl * d
        qh = q_ref[0, :, c:c + d]
        lg_lo = jax.lax.dot_general(qh, klo_ref[0, :, c:c + d], dimnums,
                                    preferred_element_type=jnp.float32)
        lg_hi = jax.lax.dot_general(qh, khi_ref[0, :, c:c + d], dimnums,
                                    preferred_element_type=jnp.float32)
        ml = jnp.concatenate([lg_lo, lg_hi], axis=1) * scale + bias
        mx = jnp.max(ml, axis=-1, keepdims=True)
        e = jnp.exp(ml - mx)
        attn = e * (1.0 / jnp.sum(e, axis=-1, keepdims=True))
        asp_ref[0, hl] = attn
        oav_ref[:, c:c + d] = _av(attn, c)
        glog = jax.lax.dot_general(qh, kg_ref[0, :, c:c + d], dimnums,
                                   preferred_element_type=jnp.float32) * scale
        gmx = jnp.max(glog, axis=-1, keepdims=True)
        ge = jnp.exp(glog - gmx)
        gsm = ge * (1.0 / jnp.sum(ge, axis=-1, keepdims=True))
        ogl_ref[:, c:c + d] = jnp.dot(gsm, vg_ref[0, :, c:c + d],
                                      preferred_element_type=jnp.float32)

    @pl.when(need_search)
    def _fixup():
        attn = jnp.reshape(asp_ref[0], (h * gs, k2))
        bits = jax.lax.bitcast_convert_type(attn, jnp.int32)
        lo0 = jnp.zeros((h * gs, 1), jnp.int32)
        hi0 = jnp.full((h * gs, 1), 0x3F800001, jnp.int32)

        def vbody(_, carry):
            lo, hi = carry
            mid = (lo + hi) // 2
            cnt = jnp.sum((bits >= mid).astype(jnp.int32), axis=-1,
                          keepdims=True)
            pred = cnt >= keep
            return jnp.where(pred, mid, lo), jnp.where(pred, hi, mid)

        vlo, _ = jax.lax.fori_loop(0, 31, vbody, (lo0, hi0))

        gt = bits > vlo
        c_gt = jnp.sum(gt.astype(jnp.int32), axis=-1, keepdims=True)
        budget = keep - c_gt
        eq = bits == vlo
        eqi = eq.astype(jnp.int32)
        lane = jax.lax.broadcasted_iota(jnp.int32, (1, k2), 1)

        tlo0 = jnp.zeros((h * gs, 1), jnp.int32)
        thi0 = jnp.full((h * gs, 1), k2 + 1, jnp.int32)

        def tbody(_, carry):
            tlo, thi = carry
            mid = (tlo + thi) // 2
            cnt = jnp.sum(jnp.where(lane < mid, eqi, 0), axis=-1,
                          keepdims=True)
            pred = cnt <= budget
            return jnp.where(pred, mid, tlo), jnp.where(pred, thi, mid)

        tlo, _ = jax.lax.fori_loop(0, 10, tbody, (tlo0, thi0))

        smask = gt | (eq & (lane < tlo))
        asp = jnp.where(smask, attn, 0.0)
        asp = asp / (jnp.sum(asp, axis=-1, keepdims=True) + 1e-9)
        asp_ref[0] = jnp.reshape(asp, (h, gs, k2))
        for hl in range(h):
            c = hl * d
            oav_ref[:, c:c + d] = _av(asp_ref[0, hl], c)

    out_ref[0] = jnp.dot(oav_ref[...] + ogl_ref[...], wpt_ref[...],
                         preferred_element_type=jnp.float32)


@jax.jit
def kernel(x, labels, scores, prototypes, Wq, Wk, Wv, Wproj):
    B, N, C = x.shape
    QK = Wq.shape[0]
    M = prototypes.shape[1]
    h, gs, d = _H, _GS, QK // _H
    ng = N // gs
    k2 = 2 * gs

    tail = jnp.flip(x[:, -gs:], axis=1)
    stream = jnp.concatenate([x, tail, prototypes], axis=1)
    NP = ((N + gs + M + 255) // 256) * 256
    stream = jnp.pad(stream, ((0, 0), (0, NP - (N + gs + M)), (0, 0)))
    w_all = jnp.concatenate([Wq, Wk, Wv], axis=0).T
    W3 = w_all.shape[1]

    qkv = pl.pallas_call(
        _qkv_kernel,
        grid=(B, NP // 256),
        in_specs=[
            pl.BlockSpec((1, 256, C), lambda b, i: (b, i, 0)),
            pl.BlockSpec((C, W3), lambda b, i: (0, 0)),
        ],
        out_specs=pl.BlockSpec((1, 256, W3), lambda b, i: (b, i, 0)),
        out_shape=jax.ShapeDtypeStruct((B, NP, W3), jnp.float32),
        compiler_params=pltpu.CompilerParams(
            dimension_semantics=("parallel", "parallel")),
    )(stream, w_all)

    labels = labels.astype(jnp.int32)
    lab_tail = jnp.flip(labels[:, -gs:], axis=1)
    labp = jnp.concatenate([labels, lab_tail], axis=1).reshape(B * (ng + 1), 1, gs)
    labq = labels.reshape(B * ng, gs, 1)
    scr = scores.reshape(B * ng, 1, gs).astype(jnp.float32)
    wpt = Wproj.T

    pg = (N + gs) // M

    in_specs = [
        pl.BlockSpec((1, gs, QK), lambda i: (i // ng, i % ng, 0)),
        pl.BlockSpec((1, gs, QK), lambda i: (i // ng, i % ng, 1)),
        pl.BlockSpec((1, gs, QK), lambda i: (i // ng, (i % ng) + 1, 1)),
        pl.BlockSpec((1, M, QK), lambda i: (i // ng, pg, 1)),
        pl.BlockSpec((1, gs, C), lambda i: (i // ng, i % ng, 2)),
        pl.BlockSpec((1, gs, C), lambda i: (i // ng, (i % ng) + 1, 2)),
        pl.BlockSpec((1, M, C), lambda i: (i // ng, pg, 2)),
        pl.BlockSpec((1, 1, gs), lambda i: ((i // ng) * (ng + 1) + i % ng, 0, 0)),
        pl.BlockSpec((1, 1, gs), lambda i: ((i // ng) * (ng + 1) + i % ng + 1, 0, 0)),
        pl.BlockSpec((1, gs, 1), lambda i: (i, 0, 0)),
        pl.BlockSpec((1, 1, gs), lambda i: (i, 0, 0)),
        pl.BlockSpec((C, C), lambda i: (0, 0)),
    ]
    out_specs = [
        pl.BlockSpec((1, gs, C), lambda i: (i // ng, i % ng, 0)),
        pl.BlockSpec((1, h, gs, k2), lambda i: (i, 0, 0, 0)),
    ]
    out_shape = [
        jax.ShapeDtypeStruct((B, N, C), jnp.float32),
        jax.ShapeDtypeStruct((B * ng, h, gs, k2), jnp.float32),
    ]

    body = functools.partial(_attn_kernel, h=h, d=d, gs=gs, nlab=_NLAB)
    out, asp = pl.pallas_call(
        body,
        grid=(B * ng,),
        in_specs=in_specs,
        out_specs=out_specs,
        out_shape=out_shape,
        scratch_shapes=[pltpu.VMEM((gs, h * d), jnp.float32),
                        pltpu.VMEM((gs, h * d), jnp.float32)],
        compiler_params=pltpu.CompilerParams(
            dimension_semantics=("parallel",)),
    )(qkv, qkv, qkv, qkv, qkv, qkv, qkv, labp, labp, labq, scr, wpt)

    return out, asp.reshape(B, ng, h, gs, k2)
